# Initial kernel scaffold; baseline (speedup 1.0000x reference)
import jax
import jax.numpy as jnp
from jax.experimental import pallas as pl


def kernel(x, edge_attr, params, edge_index, batch):
    raise NotImplementedError("write your pallas kernel here")


# speedup vs baseline: 1.8786x; 1.8786x over previous
import functools

import jax
import jax.numpy as jnp
from jax import lax
from jax.experimental import pallas as pl
from jax.experimental.pallas import tpu as pltpu
from jax.experimental.pallas import tpu_sc as plsc


def _embed_body(x_ref, emb_ref, o_ref):
    x = x_ref[...]
    na = emb_ref.shape[0]
    atomic = x[:, 0:1].astype(jnp.int32)
    oh = (atomic == lax.broadcasted_iota(jnp.int32, (x.shape[0], na), 1))
    emb = jnp.dot(oh.astype(jnp.float32), emb_ref[...],
                  preferred_element_type=jnp.float32,
                  precision=lax.Precision.HIGHEST)
    o_ref[...] = jnp.concatenate([emb, x[:, 1:]], axis=1)


def _embed(x, atom_emb):
    n, nf = x.shape
    emb_dim = atom_emb.shape[1]
    return pl.pallas_call(
        _embed_body,
        out_shape=jax.ShapeDtypeStruct((n, emb_dim + nf - 1), jnp.float32),
    )(x, atom_emb)


def _edge_mm_body(a_ref, w_ref, b_ref, o_ref):
    o_ref[...] = jnp.dot(a_ref[...].astype(jnp.bfloat16),
                         w_ref[...].astype(jnp.bfloat16),
                         preferred_element_type=jnp.float32) + b_ref[...]


def _edge_mm(edge_attr, w, b):
    e_num = edge_attr.shape[0]
    be = 5120
    grid = e_num // be
    h = w.shape[1]
    return pl.pallas_call(
        _edge_mm_body,
        grid=(grid,),
        in_specs=[
            pl.BlockSpec((be, edge_attr.shape[1]), lambda i: (i, 0)),
            pl.BlockSpec(w.shape, lambda i: (0, 0)),
            pl.BlockSpec((1, h), lambda i: (0, 0)),
        ],
        out_specs=pl.BlockSpec((be, h), lambda i: (i, 0)),
        out_shape=jax.ShapeDtypeStruct((e_num, h), jnp.float32),
    )(edge_attr, w, b.reshape(1, h))


_FLUSH = 1024
_CHUNK = 2000


def _partition(n, nc, ns):
    n2 = n // nc
    npw = (n2 // ns) & ~7
    last = n2 - (ns - 1) * npw
    return n2, npw, last


@functools.lru_cache(maxsize=None)
def _make_build_lists(n, e_num):
    info = plsc.get_sparse_core_info()
    nc, ns = info.num_cores, info.num_subcores
    nw = nc * ns
    n2, npw, last = _partition(n, nc, ns)
    cap = ((e_num + _FLUSH) // _FLUSH + 1) * _FLUSH
    nsteps = e_num // _CHUNK
    mesh = plsc.VectorSubcoreMesh(core_axis_name="c", subcore_axis_name="s")

    @functools.partial(
        pl.kernel,
        mesh=mesh,
        compiler_params=pltpu.CompilerParams(needs_layout_passes=False),
        out_type=(
            jax.ShapeDtypeStruct((nw * cap,), jnp.int32),
            jax.ShapeDtypeStruct((nw * cap,), jnp.int32),
            jax.ShapeDtypeStruct((nw * cap,), jnp.int32),
            jax.ShapeDtypeStruct((nw * 128,), jnp.int32),
        ),
        scratch_types=[
            pltpu.VMEM((_CHUNK,), jnp.int32),
            pltpu.VMEM((_CHUNK,), jnp.int32),
            pltpu.VMEM((_FLUSH + 16,), jnp.int32),
            pltpu.VMEM((_FLUSH + 16,), jnp.int32),
            pltpu.VMEM((_FLUSH + 16,), jnp.int32),
            pltpu.VMEM((128,), jnp.int32),
        ],
    )
    def build(src_hbm, dst_hbm, eid_hbm, srcl_hbm, dstl_hbm, cnt_hbm,
              dbuf, sbuf, st_eid, st_src, st_dst, cbuf):
        c = lax.axis_index("c")
        s = lax.axis_index("s")
        w = c * ns + s
        lo = c * n2 + s * npw
        my_np = jnp.where(s == ns - 1, last, npw)
        hi = lo + my_np
        iot = lax.iota(jnp.int32, 16)

        def step(ci, carry):
            fill, nflush = carry
            base = ci * _CHUNK
            pltpu.sync_copy(dst_hbm.at[pl.ds(base, _CHUNK)], dbuf)
            pltpu.sync_copy(src_hbm.at[pl.ds(base, _CHUNK)], sbuf)

            def vreg(v, carry2):
                fill, nflush = carry2
                d = dbuf[pl.ds(v * 16, 16)]
                sv = sbuf[pl.ds(v * 16, 16)]
                eidv = iot + (base + v * 16)
                msk = (d >= lo) & (d < hi)
                mi = msk.astype(jnp.int32)
                csum = plsc.cumsum(mi)
                pos = fill + csum - mi
                plsc.store_scatter(st_eid, [pos], eidv, mask=msk)
                plsc.store_scatter(st_src, [pos], sv, mask=msk)
                plsc.store_scatter(st_dst, [pos], d - lo, mask=msk)
                fill = fill + csum[15]

                def flush():
                    off = w * cap + nflush * _FLUSH
                    pltpu.sync_copy(st_eid.at[pl.ds(0, _FLUSH)],
                                    eid_hbm.at[pl.ds(off, _FLUSH)])
                    pltpu.sync_copy(st_src.at[pl.ds(0, _FLUSH)],
                                    srcl_hbm.at[pl.ds(off, _FLUSH)])
                    pltpu.sync_copy(st_dst.at[pl.ds(0, _FLUSH)],
                                    dstl_hbm.at[pl.ds(off, _FLUSH)])
                    st_eid[pl.ds(0, 16)] = st_eid[pl.ds(_FLUSH, 16)]
                    st_src[pl.ds(0, 16)] = st_src[pl.ds(_FLUSH, 16)]
                    st_dst[pl.ds(0, 16)] = st_dst[pl.ds(_FLUSH, 16)]
                pl.when(fill >= _FLUSH)(flush)
                sel = (fill >= _FLUSH).astype(jnp.int32)
                return fill - sel * _FLUSH, nflush + sel
            return lax.fori_loop(0, _CHUNK // 16, vreg, (fill, nflush))

        fill, nflush = lax.fori_loop(0, nsteps, step,
                                     (jnp.int32(0), jnp.int32(0)))

        def final_flush():
            off = w * cap + nflush * _FLUSH
            pltpu.sync_copy(st_eid.at[pl.ds(0, _FLUSH)],
                            eid_hbm.at[pl.ds(off, _FLUSH)])
            pltpu.sync_copy(st_src.at[pl.ds(0, _FLUSH)],
                            srcl_hbm.at[pl.ds(off, _FLUSH)])
            pltpu.sync_copy(st_dst.at[pl.ds(0, _FLUSH)],
                            dstl_hbm.at[pl.ds(off, _FLUSH)])
        pl.when(fill > 0)(final_flush)

        cnt = nflush * _FLUSH + fill
        cbuf[pl.ds(0, 16)] = jnp.full((16,), 1, jnp.int32) * cnt
        pltpu.sync_copy(cbuf, cnt_hbm.at[pl.ds(w * 128, 128)])

    return build


@functools.lru_cache(maxsize=None)
def _make_sc_layer(n, e_num, h):
    info = plsc.get_sparse_core_info()
    nc, ns = info.num_cores, info.num_subcores
    nw = nc * ns
    n2, npw, last = _partition(n, nc, ns)
    cap = ((e_num + _FLUSH) // _FLUSH + 1) * _FLUSH
    slab = ((last + 8) + 7) & ~7
    dump = slab - 1
    mesh = plsc.VectorSubcoreMesh(core_axis_name="c", subcore_axis_name="s")

    @functools.partial(
        pl.kernel,
        mesh=mesh,
        out_type=jax.ShapeDtypeStruct((n, h), jnp.float32),
        scratch_types=[
            pltpu.VMEM((128,), jnp.int32),
            pltpu.VMEM((128,), jnp.int32),
            pltpu.VMEM((128,), jnp.int32),
            pltpu.VMEM((128,), jnp.int32),
            pltpu.VMEM((128, h), jnp.float32),
            pltpu.VMEM((128, h), jnp.float32),
            pltpu.VMEM((slab, h), jnp.float32),
            pltpu.SemaphoreType.DMA,
        ],
    )
    def layer(h_hbm, e_hbm, eid_hbm, srcl_hbm, dstl_hbm, cnt_hbm, out_hbm,
              ebuf, sbuf, dbuf, cbuf, rows, erows, agg, sem):
        c = lax.axis_index("c")
        s = lax.axis_index("s")
        w = c * ns + s
        lo = c * n2 + s * npw
        iot = lax.iota(jnp.int32, 16)

        def zrow(i, _):
            for j in range(h // 16):
                agg[i, pl.ds(j * 16, 16)] = jnp.zeros((16,), jnp.float32)
            return 0
        lax.fori_loop(0, slab, zrow, 0)

        pltpu.sync_copy(cnt_hbm.at[pl.ds(w * 128, 128)], cbuf)
        cnt = cbuf[pl.ds(0, 16)][0]
        nchunks = lax.div(cnt + 127, 128)

        def chunk(i, _):
            off = w * cap + i * 128
            pltpu.sync_copy(eid_hbm.at[pl.ds(off, 128)], ebuf)
            pltpu.sync_copy(srcl_hbm.at[pl.ds(off, 128)], sbuf)
            pltpu.sync_copy(dstl_hbm.at[pl.ds(off, 128)], dbuf)
            limit = cnt - i * 128
            for v in range(8):
                msk = (iot + (v * 16)) < jnp.full((16,), 1, jnp.int32) * limit
                sl = pl.ds(v * 16, 16)
                ebuf[sl] = jnp.where(msk, ebuf[sl], 0)
                sbuf[sl] = jnp.where(msk, sbuf[sl], 0)
                dbuf[sl] = jnp.where(msk, dbuf[sl], dump)
            cp0 = pltpu.async_copy(h_hbm.at[sbuf], rows, sem)
            cp1 = pltpu.async_copy(e_hbm.at[ebuf], erows, sem)
            cp0.wait()
            cp1.wait()

            def pgrp(g, _):
                dv = dbuf[pl.ds(g * 16, 16)]
                for i in range(16):
                    r = g * 16 + i
                    dl = dv[i]
                    for j in range(h // 16):
                        sl = pl.ds(j * 16, 16)
                        m = jnp.maximum(rows[r, sl] + erows[r, sl], 0.0)
                        agg[dl, sl] = agg[dl, sl] + m
                return 0
            lax.fori_loop(0, 8, pgrp, 0)
            return 0
        lax.fori_loop(0, nchunks, chunk, 0)

        pltpu.sync_copy(agg.at[pl.ds(0, npw)], out_hbm.at[pl.ds(lo, npw)])

        def last_rows():
            pltpu.sync_copy(agg.at[pl.ds(npw, last - npw)],
                            out_hbm.at[pl.ds(lo + npw, last - npw)])
        pl.when(s == ns - 1)(last_rows)

    return layer

def _node_body(h_ref, agg_ref, wa_ref, ba_ref, wb_ref, bb_ref, g_ref, bt_ref,
               o_ref, zs_ref):
    z = h_ref[...] + agg_ref[...]
    z1 = jax.nn.relu(jnp.dot(z.astype(jnp.bfloat16),
                             wa_ref[...].astype(jnp.bfloat16),
                             preferred_element_type=jnp.float32) + ba_ref[...])
    z2 = jnp.dot(z1.astype(jnp.bfloat16), wb_ref[...].astype(jnp.bfloat16),
                 preferred_element_type=jnp.float32) + bb_ref[...]
    n, hh = z2.shape

    def _colsum():
        def _acc(t, a):
            return a + zs_ref[pl.ds(t * 8, 8), :]
        acc = lax.fori_loop(0, n // 8, _acc, jnp.zeros((8, hh), jnp.float32))
        for _ in range(3):
            k = acc.shape[0] // 2
            a2 = acc.reshape(k, 2, hh)
            acc = a2[:, 0, :] + a2[:, 1, :]
        return acc

    rcp = jnp.float32(1.0 / n)
    zs_ref[...] = z2
    mu = _colsum() * rcp
    d = z2 - mu
    zs_ref[...] = d * d
    var = _colsum() * rcp
    o_ref[...] = jax.nn.relu(d / jnp.sqrt(var + 1e-5) * g_ref[...]
                             + bt_ref[...])


def _node_update(hmat, agg, wa, ba, wb, bb, g, bt):
    n, h = hmat.shape
    return pl.pallas_call(
        _node_body,
        out_shape=jax.ShapeDtypeStruct((n, h), jnp.float32),
        scratch_shapes=[pltpu.VMEM((n, h), jnp.float32)],
    )(hmat, agg, wa, ba.reshape(1, h), wb, bb.reshape(1, h),
      g.reshape(1, h), bt.reshape(1, h))


def _pool_body(h_ref, b_ref, w1_ref, b1_ref, w2_ref, b2_ref, o_ref):
    n = h_ref.shape[0]
    g = o_ref.shape[0]
    oh = (b_ref[...] == lax.broadcasted_iota(jnp.int32, (n, g), 1))
    oh = oh.astype(jnp.float32)
    dnums = (((0,), (0,)), ((), ()))
    sums = lax.dot_general(oh, h_ref[...], dnums,
                           preferred_element_type=jnp.float32,
                           precision=lax.Precision.HIGHEST)
    cnt = lax.dot_general(oh, jnp.ones((n, 1), jnp.float32), dnums,
                          preferred_element_type=jnp.float32,
                          precision=lax.Precision.HIGHEST)
    graph = sums / jnp.maximum(cnt, 1.0)
    z1 = jax.nn.relu(jnp.dot(graph.astype(jnp.bfloat16),
                             w1_ref[...].astype(jnp.bfloat16),
                             preferred_element_type=jnp.float32) + b1_ref[...])
    o_ref[...] = jnp.dot(z1.astype(jnp.bfloat16), w2_ref[...].astype(jnp.bfloat16),
                         preferred_element_type=jnp.float32) + b2_ref[...]


def _pool_mlp(hmat, batch2d, num_graphs, w1, b1, w2, b2):
    h = hmat.shape[1]
    return pl.pallas_call(
        _pool_body,
        out_shape=jax.ShapeDtypeStruct((num_graphs, 1), jnp.float32),
    )(hmat, batch2d, w1, b1.reshape(1, h), w2, b2.reshape(1, 1))


def kernel(x, edge_attr, params, edge_index, batch):
    n = x.shape[0]
    e_num = edge_attr.shape[0]
    num_graphs = 256

    hmat = _embed(x, params['atom_emb'])
    h = hmat.shape[1]

    src1 = edge_index[0].reshape(e_num)
    dst1 = edge_index[1].reshape(e_num)
    build = _make_build_lists(n, e_num)
    eids, srcl, dstl, cnts = build(src1, dst1)
    sc_layer = _make_sc_layer(n, e_num, h)

    for l in range(4):
        e = _edge_mm(edge_attr, params['We%d' % l], params['be%d' % l])
        agg = sc_layer(hmat, e, eids, srcl, dstl, cnts)
        hmat = _node_update(hmat, agg,
                            params['W%da' % l], params['b%da' % l],
                            params['W%db' % l], params['b%db' % l],
                            params['g%d' % l], params['bt%d' % l])

    batch2d = batch.reshape(n, 1)
    out = _pool_mlp(hmat, batch2d, num_graphs,
                    params['Wm1'], params['bm1'], params['Wm2'], params['bm2'])
    return out.reshape(-1)
